# baseline (speedup 1.0000x reference)
import functools

import jax
import jax.numpy as jnp
from jax import lax
from jax.experimental import pallas as pl
from jax.experimental.pallas import tpu as pltpu
from jax.experimental.pallas import tpu_sc as plsc

NC = 2
NS = 16
L = 16


def _make_tc_transpose(N, D, dtype):
    RN = 1024

    def body(a_ref, at_ref):
        at_ref[...] = a_ref[...].T

    return pl.pallas_call(
        body,
        grid=(pl.cdiv(N, RN),),
        in_specs=[pl.BlockSpec((RN, D), lambda i: (i, 0))],
        out_specs=pl.BlockSpec((D, RN), lambda i: (0, i)),
        out_shape=jax.ShapeDtypeStruct((D, N), dtype),
    )


def _make_sc_kernel(M, D, B):
    total = M * D
    NCHUNK = 4
    assert D == NCHUNK * NS
    CW = NS * M
    NPASS = NCHUNK // NC
    PW = CW // NS
    assert PW % 8 == 0
    BLK = 4096
    assert B % BLK == 0
    NBLK = B // BLK
    NVEC = BLK // L

    mesh = plsc.VectorSubcoreMesh(core_axis_name="c", subcore_axis_name="s")

    @functools.partial(
        pl.kernel,
        mesh=mesh,
        out_type=jax.ShapeDtypeStruct((total,), jnp.float32),
        compiler_params=pltpu.CompilerParams(use_tc_tiling_on_sc=False),
        scratch_types=[
            pltpu.VMEM_SHARED((CW + 16,), jnp.float32),
            pltpu.VMEM((BLK,), jnp.int32),
            pltpu.VMEM((BLK,), jnp.int32),
            pltpu.VMEM((BLK,), jnp.float32),
            pltpu.VMEM((BLK,), jnp.float32),
            pltpu.VMEM((BLK,), jnp.int32),
            pltpu.VMEM((BLK,), jnp.int32),
            pltpu.SemaphoreType.DMA,
            pltpu.SemaphoreType.DMA,
        ],
    )
    def scatter_add_kernel(xt_hbm, idxt_hbm, srct_hbm, outt_hbm,
                           accum, idx_raw0, idx_raw1, src_buf0, src_buf1,
                           idx_scat0, idx_scat1, sem, lsem):
        idx_raw = (idx_raw0, idx_raw1)
        src_buf = (src_buf0, src_buf1)
        idx_scat = (idx_scat0, idx_scat1)
        c = lax.axis_index("c")
        s = lax.axis_index("s")

        for p in range(NPASS):
            k = p * NC + c
            base = k * CW
            colbase = (k * NS + s) * B

            pltpu.sync_copy(xt_hbm.at[pl.ds(base + s * PW, PW)],
                            accum.at[pl.ds(s * PW, PW)])
            plsc.subcore_barrier()

            def islice(b):
                return idxt_hbm.at[pl.ds(colbase + b * BLK, BLK)]

            def sslice(b):
                return srct_hbm.at[pl.ds(colbase + b * BLK, BLK)]

            pltpu.async_copy(islice(0), idx_raw[0], lsem)
            pltpu.async_copy(sslice(0), src_buf[0], lsem)
            for b in range(NBLK):
                d = b % 2
                pltpu.make_async_copy(islice(b), idx_raw[d], lsem).wait()
                pltpu.make_async_copy(sslice(b), src_buf[d], lsem).wait()

                def vec_body(i, _, d=d):
                    v = idx_raw[d][pl.ds(i * L, L)]
                    idx_scat[d][pl.ds(i * L, L)] = v + s * M
                    return 0

                lax.fori_loop(0, NVEC, vec_body, 0)
                if b >= 1:
                    pltpu.make_async_copy(src_buf[1 - d],
                                          accum.at[idx_scat[1 - d]],
                                          sem).wait()
                if b + 1 < NBLK:
                    pltpu.async_copy(islice(b + 1), idx_raw[1 - d], lsem)
                    pltpu.async_copy(sslice(b + 1), src_buf[1 - d], lsem)
                pltpu.async_copy(src_buf[d], accum.at[idx_scat[d]],
                                 sem, add=True)
            pltpu.make_async_copy(src_buf[(NBLK - 1) % 2],
                                  accum.at[idx_scat[(NBLK - 1) % 2]],
                                  sem).wait()
            plsc.subcore_barrier()

            pltpu.sync_copy(accum.at[pl.ds(s * PW, PW)],
                            outt_hbm.at[pl.ds(base + s * PW, PW)])
            plsc.subcore_barrier()

    return scatter_add_kernel


def kernel(x, dim, index, src, out):
    M, D = x.shape
    B = src.shape[0]
    del out
    rows = index + jnp.asarray(dim, dtype=index.dtype)
    sc = _make_sc_kernel(M, D, B)
    xt = _make_tc_transpose(M, D, x.dtype)(x)
    outt = sc(xt.reshape(-1), rows.T.reshape(-1), src.T.reshape(-1))
    return outt.reshape(D, M).T


# speedup vs baseline: 1.4216x; 1.4216x over previous
import functools

import jax
import jax.numpy as jnp
from jax import lax
from jax.experimental import pallas as pl
from jax.experimental.pallas import tpu as pltpu
from jax.experimental.pallas import tpu_sc as plsc

NC = 2
NS = 16
L = 16


def _make_sc_kernel(M, D, B):
    total = M * D
    NCHUNK = 4
    assert D == NCHUNK * NS
    CW = NS * M
    NPASS = NCHUNK // NC
    PW = CW // NS
    assert PW % 8 == 0
    RPT = B // NS
    RB = 128
    assert RPT % RB == 0
    NBLK = RPT // RB

    mesh = plsc.VectorSubcoreMesh(core_axis_name="c", subcore_axis_name="s")

    @functools.partial(
        pl.kernel,
        mesh=mesh,
        out_type=jax.ShapeDtypeStruct((total,), jnp.float32),
        compiler_params=pltpu.CompilerParams(use_tc_tiling_on_sc=False),
        scratch_types=[
            pltpu.VMEM_SHARED((CW + 16,), jnp.float32),
            pltpu.VMEM((RB, 16), jnp.int32),
            pltpu.VMEM((RB, 16), jnp.int32),
            pltpu.VMEM((RB, 16), jnp.float32),
            pltpu.VMEM((RB, 16), jnp.float32),
            pltpu.VMEM((RB * 16,), jnp.int32),
            pltpu.VMEM((RB * 16,), jnp.int32),
            pltpu.VMEM((RB * 16,), jnp.float32),
            pltpu.VMEM((RB * 16,), jnp.float32),
            pltpu.SemaphoreType.DMA,
            pltpu.SemaphoreType.DMA,
        ],
    )
    def scatter_add_kernel(xt_hbm, idx_hbm, src_hbm, outt_hbm,
                           accum, idx_raw0, idx_raw1, src_buf0, src_buf1,
                           idx_scat0, idx_scat1, val0, val1, sem, lsem):
        idx_raw = (idx_raw0, idx_raw1)
        src_buf = (src_buf0, src_buf1)
        idx_scat = (idx_scat0, idx_scat1)
        val = (val0, val1)
        c = lax.axis_index("c")
        s = lax.axis_index("s")
        iota = lax.iota(jnp.int32, L)
        iota_m = iota * M

        for p in range(NPASS):
            k = p * NC + c
            base = k * CW
            k16 = k * NS
            row0 = s * RPT

            pltpu.sync_copy(xt_hbm.at[pl.ds(base + s * PW, PW)],
                            accum.at[pl.ds(s * PW, PW)])
            plsc.subcore_barrier()

            def islice(b):
                return idx_hbm.at[pl.ds(row0 + b * RB, RB), pl.ds(k16, 16)]

            def sslice(b):
                return src_hbm.at[pl.ds(row0 + b * RB, RB), pl.ds(k16, 16)]

            pltpu.async_copy(islice(0), idx_raw[0], lsem)
            pltpu.async_copy(sslice(0), src_buf[0], lsem)
            for b in range(NBLK):
                d = b % 2
                pltpu.make_async_copy(islice(b), idx_raw[d], lsem).wait()
                pltpu.make_async_copy(sslice(b), src_buf[d], lsem).wait()

                def vec_body(i, _, d=d):
                    v = idx_raw[d][i, :]
                    idx_scat[d][pl.ds(i * L, L)] = iota_m + v
                    val[d][pl.ds(i * L, L)] = src_buf[d][i, :]
                    return 0

                lax.fori_loop(0, RB, vec_body, 0)
                if b >= 1:
                    pltpu.make_async_copy(val[1 - d],
                                          accum.at[idx_scat[1 - d]],
                                          sem).wait()
                if b + 1 < NBLK:
                    pltpu.async_copy(islice(b + 1), idx_raw[1 - d], lsem)
                    pltpu.async_copy(sslice(b + 1), src_buf[1 - d], lsem)
                pltpu.async_copy(val[d], accum.at[idx_scat[d]],
                                 sem, add=True)
            pltpu.make_async_copy(val[(NBLK - 1) % 2],
                                  accum.at[idx_scat[(NBLK - 1) % 2]],
                                  sem).wait()
            plsc.subcore_barrier()

            pltpu.sync_copy(accum.at[pl.ds(s * PW, PW)],
                            outt_hbm.at[pl.ds(base + s * PW, PW)])
            plsc.subcore_barrier()

    return scatter_add_kernel


def kernel(x, dim, index, src, out):
    M, D = x.shape
    B = src.shape[0]
    del out
    rows = index + jnp.asarray(dim, dtype=index.dtype)
    sc = _make_sc_kernel(M, D, B)
    outt = sc(x.T.reshape(-1), rows, src)
    return outt.reshape(D, M).T


# speedup vs baseline: 1.7980x; 1.2648x over previous
import functools

import jax
import jax.numpy as jnp
from jax import lax
from jax.experimental import pallas as pl
from jax.experimental.pallas import tpu as pltpu
from jax.experimental.pallas import tpu_sc as plsc

NC = 2
NS = 16
L = 16


def _make_sc_kernel(M, D, B):
    total = M * D
    NCHUNK = 4
    assert D == NCHUNK * NS
    CW = NS * M
    NPASS = NCHUNK // NC
    PW = CW // NS
    assert PW % 8 == 0
    BLK = 4096
    assert B % BLK == 0
    NBLK = B // BLK
    NVEC = BLK // L

    mesh = plsc.VectorSubcoreMesh(core_axis_name="c", subcore_axis_name="s")

    @functools.partial(
        pl.kernel,
        mesh=mesh,
        out_type=jax.ShapeDtypeStruct((total,), jnp.float32),
        compiler_params=pltpu.CompilerParams(use_tc_tiling_on_sc=False),
        scratch_types=[
            pltpu.VMEM_SHARED((CW + 16,), jnp.float32),
            pltpu.VMEM((BLK,), jnp.int32),
            pltpu.VMEM((BLK,), jnp.int32),
            pltpu.VMEM((BLK,), jnp.float32),
            pltpu.VMEM((BLK,), jnp.float32),
            pltpu.VMEM((BLK,), jnp.int32),
            pltpu.VMEM((BLK,), jnp.int32),
            pltpu.SemaphoreType.DMA,
            pltpu.SemaphoreType.DMA,
        ],
    )
    def scatter_add_kernel(xt_hbm, idxt_hbm, srct_hbm, outt_hbm,
                           accum, idx_raw0, idx_raw1, src_buf0, src_buf1,
                           idx_scat0, idx_scat1, sem, lsem):
        idx_raw = (idx_raw0, idx_raw1)
        src_buf = (src_buf0, src_buf1)
        idx_scat = (idx_scat0, idx_scat1)
        c = lax.axis_index("c")
        s = lax.axis_index("s")

        for p in range(NPASS):
            k = p * NC + c
            base = k * CW
            colbase = (k * NS + s) * B

            def islice(b):
                return idxt_hbm.at[pl.ds(colbase + b * BLK, BLK)]

            def sslice(b):
                return srct_hbm.at[pl.ds(colbase + b * BLK, BLK)]

            pltpu.async_copy(islice(0), idx_raw[0], lsem)
            pltpu.async_copy(sslice(0), src_buf[0], lsem)

            pltpu.sync_copy(xt_hbm.at[pl.ds(base + s * PW, PW)],
                            accum.at[pl.ds(s * PW, PW)])
            plsc.subcore_barrier()

            for b in range(NBLK):
                d = b % 2
                pltpu.make_async_copy(islice(b), idx_raw[d], lsem).wait()
                pltpu.make_async_copy(sslice(b), src_buf[d], lsem).wait()

                def vec_body(i, _, d=d):
                    v = idx_raw[d][pl.ds(i * L, L)]
                    idx_scat[d][pl.ds(i * L, L)] = v + s * M
                    return 0

                lax.fori_loop(0, NVEC, vec_body, 0)
                if b >= 1:
                    pltpu.make_async_copy(src_buf[1 - d],
                                          accum.at[idx_scat[1 - d]],
                                          sem).wait()
                if b + 1 < NBLK:
                    pltpu.async_copy(islice(b + 1), idx_raw[1 - d], lsem)
                    pltpu.async_copy(sslice(b + 1), src_buf[1 - d], lsem)
                pltpu.async_copy(src_buf[d], accum.at[idx_scat[d]],
                                 sem, add=True)
            pltpu.make_async_copy(src_buf[(NBLK - 1) % 2],
                                  accum.at[idx_scat[(NBLK - 1) % 2]],
                                  sem).wait()
            plsc.subcore_barrier()

            pltpu.sync_copy(accum.at[pl.ds(s * PW, PW)],
                            outt_hbm.at[pl.ds(base + s * PW, PW)])

    return scatter_add_kernel


def kernel(x, dim, index, src, out):
    M, D = x.shape
    B = src.shape[0]
    del out
    rows = index + jnp.asarray(dim, dtype=index.dtype)
    sc = _make_sc_kernel(M, D, B)
    outt = sc(x.T.reshape(-1), rows.T.reshape(-1), src.T.reshape(-1))
    return outt.reshape(D, M).T


# speedup vs baseline: 1.8015x; 1.0019x over previous
import functools

import jax
import jax.numpy as jnp
from jax import lax
from jax.experimental import pallas as pl
from jax.experimental.pallas import tpu as pltpu
from jax.experimental.pallas import tpu_sc as plsc

NC = 2
NS = 16
L = 16


def _make_sc_kernel(M, D, B):
    total = M * D
    NCHUNK = 4
    assert D == NCHUNK * NS
    CW = NS * M
    NPASS = NCHUNK // NC
    PW = CW // NS
    assert PW % 8 == 0
    BLK = 4096
    assert B % BLK == 0
    NBLK = B // BLK
    assert NBLK % 2 == 0
    NVEC = BLK // L

    mesh = plsc.VectorSubcoreMesh(core_axis_name="c", subcore_axis_name="s")

    @functools.partial(
        pl.kernel,
        mesh=mesh,
        out_type=jax.ShapeDtypeStruct((total,), jnp.float32),
        compiler_params=pltpu.CompilerParams(use_tc_tiling_on_sc=False),
        scratch_types=[
            pltpu.VMEM_SHARED((CW + 16,), jnp.float32),
            pltpu.VMEM((BLK,), jnp.int32),
            pltpu.VMEM((BLK,), jnp.int32),
            pltpu.VMEM((BLK,), jnp.float32),
            pltpu.VMEM((BLK,), jnp.float32),
            pltpu.VMEM((BLK,), jnp.int32),
            pltpu.VMEM((BLK,), jnp.int32),
            pltpu.SemaphoreType.DMA,
            pltpu.SemaphoreType.DMA,
        ],
    )
    def scatter_add_kernel(xt_hbm, idxt_hbm, srct_hbm, outt_hbm,
                           accum, idx_raw0, idx_raw1, src_buf0, src_buf1,
                           idx_scat0, idx_scat1, sem, lsem):
        idx_raw = (idx_raw0, idx_raw1)
        src_buf = (src_buf0, src_buf1)
        idx_scat = (idx_scat0, idx_scat1)
        c = lax.axis_index("c")
        s = lax.axis_index("s")

        def mkslices(p):
            k = p * NC + c
            colbase = (k * NS + s) * B

            def islice(b):
                return idxt_hbm.at[pl.ds(colbase + b * BLK, BLK)]

            def sslice(b):
                return srct_hbm.at[pl.ds(colbase + b * BLK, BLK)]

            return islice, sslice

        islice0, sslice0 = mkslices(0)
        pltpu.async_copy(islice0(0), idx_raw[0], lsem)
        pltpu.async_copy(sslice0(0), src_buf[0], lsem)

        for p in range(NPASS):
            k = p * NC + c
            base = k * CW
            islice, sslice = mkslices(p)

            pltpu.sync_copy(xt_hbm.at[pl.ds(base + s * PW, PW)],
                            accum.at[pl.ds(s * PW, PW)])
            plsc.subcore_barrier()

            for b in range(NBLK):
                d = b % 2
                pltpu.make_async_copy(islice(b), idx_raw[d], lsem).wait()
                pltpu.make_async_copy(sslice(b), src_buf[d], lsem).wait()

                def vec_body(i, _, d=d):
                    v = idx_raw[d][pl.ds(i * L, L)]
                    idx_scat[d][pl.ds(i * L, L)] = v + s * M
                    return 0

                lax.fori_loop(0, NVEC, vec_body, 0)
                if b >= 1:
                    pltpu.make_async_copy(src_buf[1 - d],
                                          accum.at[idx_scat[1 - d]],
                                          sem).wait()
                if b + 1 < NBLK:
                    pltpu.async_copy(islice(b + 1), idx_raw[1 - d], lsem)
                    pltpu.async_copy(sslice(b + 1), src_buf[1 - d], lsem)
                pltpu.async_copy(src_buf[d], accum.at[idx_scat[d]],
                                 sem, add=True)
            pltpu.make_async_copy(src_buf[(NBLK - 1) % 2],
                                  accum.at[idx_scat[(NBLK - 1) % 2]],
                                  sem).wait()
            if p + 1 < NPASS:
                islice_n, sslice_n = mkslices(p + 1)
                pltpu.async_copy(islice_n(0), idx_raw[0], lsem)
                pltpu.async_copy(sslice_n(0), src_buf[0], lsem)
            plsc.subcore_barrier()

            pltpu.sync_copy(accum.at[pl.ds(s * PW, PW)],
                            outt_hbm.at[pl.ds(base + s * PW, PW)])

    return scatter_add_kernel


def kernel(x, dim, index, src, out):
    M, D = x.shape
    B = src.shape[0]
    del out
    rows = index + jnp.asarray(dim, dtype=index.dtype)
    sc = _make_sc_kernel(M, D, B)
    outt = sc(x.T.reshape(-1), rows.T.reshape(-1), src.T.reshape(-1))
    return outt.reshape(D, M).T
